# baseline (speedup 1.0000x reference)
import functools

import jax
import jax.numpy as jnp
from jax.experimental import pallas as pl

E = 8
K = 2
FCHUNK = 512


def _routing_weights(gating, e):
    t, n = gating.shape
    m = jnp.max(gating, axis=1, keepdims=True)
    p = jnp.exp(gating - m)
    rw = p / jnp.sum(p, axis=1, keepdims=True)
    colid = jax.lax.broadcasted_iota(jnp.int32, rw.shape, 1)
    m1 = jnp.max(rw, axis=1, keepdims=True)
    i1 = jnp.min(jnp.where(rw == m1, colid, n), axis=1, keepdims=True)
    is1 = colid == i1
    rw_m = jnp.where(is1, -jnp.inf, rw)
    m2 = jnp.max(rw_m, axis=1, keepdims=True)
    i2 = jnp.min(jnp.where(rw_m == m2, colid, n), axis=1, keepdims=True)
    sel = is1 | (colid == i2)
    wmat = jnp.where(sel, rw, 0.0) / (m1 + m2)
    return jnp.sum(jnp.where(colid == e, wmat, 0.0), axis=1, keepdims=True)


def _moe_body(x_ref, gating_ref, w13g_ref, w13u_ref, w2_ref, out_ref):
    e = pl.program_id(0)
    f = pl.program_id(1)
    wcol = _routing_weights(gating_ref[...], e)

    xt = x_ref[...].astype(jnp.bfloat16)
    g = jax.lax.dot_general(xt, w13g_ref[0].astype(jnp.bfloat16),
                            (((1,), (1,)), ((), ())),
                            preferred_element_type=jnp.float32)
    u = jax.lax.dot_general(xt, w13u_ref[0].astype(jnp.bfloat16),
                            (((1,), (1,)), ((), ())),
                            preferred_element_type=jnp.float32)
    h = g * jax.nn.sigmoid(g) * u
    y = jax.lax.dot_general(h.astype(jnp.bfloat16), w2_ref[0].astype(jnp.bfloat16),
                            (((1,), (1,)), ((), ())),
                            preferred_element_type=jnp.float32)
    contrib = y * wcol

    @pl.when((e == 0) & (f == 0))
    def _():
        out_ref[...] = contrib

    @pl.when((e > 0) | (f > 0))
    def _():
        out_ref[...] = out_ref[...] + contrib


@functools.partial(jax.jit, static_argnames=())
def kernel(x, gating_output, w13, w2):
    T, D = x.shape
    F = w2.shape[2]
    nf = F // FCHUNK
    out = pl.pallas_call(
        _moe_body,
        grid=(E, nf),
        in_specs=[
            pl.BlockSpec((T, D), lambda e, f: (0, 0)),
            pl.BlockSpec((T, E), lambda e, f: (0, 0)),
            pl.BlockSpec((1, FCHUNK, D), lambda e, f: (e, f, 0)),
            pl.BlockSpec((1, FCHUNK, D), lambda e, f: (e, nf + f, 0)),
            pl.BlockSpec((1, D, FCHUNK), lambda e, f: (e, 0, f)),
        ],
        out_specs=pl.BlockSpec((T, D), lambda e, f: (0, 0)),
        out_shape=jax.ShapeDtypeStruct((T, D), jnp.float32),
    )(x, gating_output, w13, w13, w2)
    return out


# speedup vs baseline: 1.1657x; 1.1657x over previous
import functools

import jax
import jax.numpy as jnp
from jax.experimental import pallas as pl
from jax.experimental.pallas import tpu as pltpu

E = 8
K = 2
FCHUNK = 1024
RT = 512


def _routing_weights(gating):
    t, n = gating.shape
    m = jnp.max(gating, axis=1, keepdims=True)
    p = jnp.exp(gating - m)
    rw = p / jnp.sum(p, axis=1, keepdims=True)
    colid = jax.lax.broadcasted_iota(jnp.int32, rw.shape, 1)
    m1 = jnp.max(rw, axis=1, keepdims=True)
    i1 = jnp.min(jnp.where(rw == m1, colid, n), axis=1, keepdims=True)
    is1 = colid == i1
    rw_m = jnp.where(is1, -jnp.inf, rw)
    m2 = jnp.max(rw_m, axis=1, keepdims=True)
    i2 = jnp.min(jnp.where(rw_m == m2, colid, n), axis=1, keepdims=True)
    sel = is1 | (colid == i2)
    return jnp.where(sel, rw, 0.0) / (m1 + m2)


def _moe_body(x_ref, gating_ref, w13g_ref, w13u_ref, w2_ref, out_ref, wmat_ref):
    e = pl.program_id(0)
    f = pl.program_id(1)

    @pl.when((e == 0) & (f == 0))
    def _():
        wmat_ref[...] = _routing_weights(gating_ref[...])

    colid = jax.lax.broadcasted_iota(jnp.int32, wmat_ref.shape, 1)
    wcol = jnp.sum(jnp.where(colid == e, wmat_ref[...], 0.0), axis=1,
                   keepdims=True)

    wg = w13g_ref[0].astype(jnp.bfloat16)
    wu = w13u_ref[0].astype(jnp.bfloat16)
    wd = w2_ref[0].astype(jnp.bfloat16)
    nrt = x_ref.shape[0] // RT
    for i in range(nrt):
        sl = pl.ds(i * RT, RT)
        xt = x_ref[sl, :]
        g = jax.lax.dot_general(xt, wg, (((1,), (1,)), ((), ())),
                                preferred_element_type=jnp.float32)
        u = jax.lax.dot_general(xt, wu, (((1,), (1,)), ((), ())),
                                preferred_element_type=jnp.float32)
        h = (g * jax.nn.sigmoid(g) * u).astype(jnp.bfloat16)
        y = jax.lax.dot_general(h, wd, (((1,), (1,)), ((), ())),
                                preferred_element_type=jnp.float32)
        contrib = y * wcol[i * RT:(i + 1) * RT, :]

        @pl.when((e == 0) & (f == 0))
        def _():
            out_ref[sl, :] = contrib

        @pl.when((e > 0) | (f > 0))
        def _():
            out_ref[sl, :] = out_ref[sl, :] + contrib


@functools.partial(jax.jit, static_argnames=())
def kernel(x, gating_output, w13, w2):
    T, D = x.shape
    F = w2.shape[2]
    nf = F // FCHUNK
    xb = x.astype(jnp.bfloat16)
    out = pl.pallas_call(
        _moe_body,
        grid=(E, nf),
        in_specs=[
            pl.BlockSpec((T, D), lambda e, f: (0, 0)),
            pl.BlockSpec((T, E), lambda e, f: (0, 0)),
            pl.BlockSpec((1, FCHUNK, D), lambda e, f: (e, f, 0)),
            pl.BlockSpec((1, FCHUNK, D), lambda e, f: (e, nf + f, 0)),
            pl.BlockSpec((1, D, FCHUNK), lambda e, f: (e, 0, f)),
        ],
        out_specs=pl.BlockSpec((T, D), lambda e, f: (0, 0)),
        out_shape=jax.ShapeDtypeStruct((T, D), jnp.float32),
        scratch_shapes=[pltpu.VMEM((T, E), jnp.float32)],
    )(xb, gating_output, w13, w13, w2)
    return out
